# baseline (speedup 1.0000x reference)
import functools

import jax
import jax.numpy as jnp
from jax import lax
from jax.experimental import pallas as pl
from jax.experimental.pallas import tpu as pltpu
from jax.experimental.pallas import tpu_sc as plsc

N = 10000
E = 320000
D = 128
H = 128
G = 16
C = 10
EPS = 1e-5
HD = D // 2

NPAD = 10240
EPAD = 327680
EROWS = EPAD // 128
CHUNKS_W = EROWS // 16
DEG_CH_W = EROWS // 32
ROWS_W = 640
BLK = 1280
GRID = NPAD // BLK
NBUF = 2

_mesh = plsc.VectorSubcoreMesh(core_axis_name="c", subcore_axis_name="s")


@functools.partial(
    pl.kernel,
    out_type=jax.ShapeDtypeStruct((2, NPAD, HD), jnp.float32),
    mesh=_mesh,
    scratch_types=[
        pltpu.VMEM((CHUNKS_W, 128), jnp.int32),
        pltpu.VMEM((CHUNKS_W, 128), jnp.int32),
    ] + [pltpu.VMEM((128, HD), jnp.float32) for _ in range(2 * NBUF)] + [
        pltpu.VMEM_SHARED((NPAD, HD), jnp.float32),
        pltpu.SemaphoreType.DMA,
        pltpu.SemaphoreType.DMA,
    ],
    compiler_params=pltpu.CompilerParams(use_tc_tiling_on_sc=False),
)
def _sc_agg(u_hbm, src_hbm, dst_hbm, out_hbm, src_v, dst_v, b0, b1, b2, b3,
            acc, sem_g, sem_s):
    bufs = (b0, b1, b2, b3)
    c = lax.axis_index("c")
    s = lax.axis_index("s")
    pltpu.sync_copy(src_hbm.at[pl.ds(s * CHUNKS_W, CHUNKS_W)], src_v)
    pltpu.sync_copy(dst_hbm.at[pl.ds(s * CHUNKS_W, CHUNKS_W)], dst_v)
    pltpu.sync_copy(u_hbm.at[c, pl.ds(N + 48, 128)], bufs[0])
    rbase = s * ROWS_W
    for i in range(ROWS_W // 128):
        pltpu.sync_copy(bufs[0], acc.at[pl.ds(rbase + i * 128, 128)])
    for b in range(NBUF):
        pltpu.async_copy(u_hbm.at[c].at[src_v.at[b]], bufs[b], sem_g)
    plsc.subcore_barrier()

    def group(g, carry):
        for b in range(2 * NBUF):
            j = g * 2 * NBUF + b
            pltpu.make_async_copy(u_hbm.at[c].at[src_v.at[j]], bufs[b],
                                  sem_g).wait()
            pltpu.async_copy(bufs[b], acc.at[dst_v.at[j]], sem_s, add=True)
            prev = j - NBUF
            pb = (b + NBUF) % (2 * NBUF)

            @pl.when(prev >= 0)
            def _():
                pltpu.make_async_copy(bufs[pb], acc.at[dst_v.at[prev]],
                                      sem_s).wait()

            nxt = j + NBUF

            @pl.when(nxt < CHUNKS_W)
            def _():
                pltpu.async_copy(u_hbm.at[c].at[src_v.at[nxt]], bufs[pb],
                                 sem_g)
        return carry

    lax.fori_loop(0, CHUNKS_W // (2 * NBUF), group, 0)
    for k in range(NBUF):
        j = CHUNKS_W - NBUF + k
        pltpu.make_async_copy(bufs[(j % (2 * NBUF))],
                              acc.at[dst_v.at[j]], sem_s).wait()
    plsc.subcore_barrier()
    for i in range(ROWS_W // 128):
        pltpu.sync_copy(acc.at[pl.ds(rbase + i * 128, 128)], bufs[0])
        pltpu.sync_copy(bufs[0], out_hbm.at[c, pl.ds(rbase + i * 128, 128)])


@functools.partial(
    pl.kernel,
    out_type=jax.ShapeDtypeStruct((2, NPAD), jnp.float32),
    mesh=_mesh,
    scratch_types=[
        pltpu.VMEM((DEG_CH_W, 128), jnp.int32),
        pltpu.VMEM((128,), jnp.float32),
        pltpu.VMEM((ROWS_W,), jnp.float32),
        pltpu.VMEM_SHARED((NPAD,), jnp.float32),
    ],
)
def _sc_deg(dst_hbm, ones_hbm, zeros_hbm, out_hbm, dst_v, ones_v, zeros_v,
            acc):
    c = lax.axis_index("c")
    s = lax.axis_index("s")
    w = c * 16 + s
    pltpu.sync_copy(dst_hbm.at[pl.ds(w * DEG_CH_W, DEG_CH_W)], dst_v)
    pltpu.sync_copy(ones_hbm, ones_v)
    pltpu.sync_copy(zeros_hbm, zeros_v)
    pltpu.sync_copy(zeros_v, acc.at[pl.ds(s * ROWS_W, ROWS_W)])
    plsc.subcore_barrier()

    def body(j, carry):
        pltpu.sync_copy(ones_v, acc.at[dst_v.at[j]], add=True)
        return carry

    lax.fori_loop(0, DEG_CH_W, body, 0)
    plsc.subcore_barrier()
    pltpu.sync_copy(acc.at[pl.ds(s * ROWS_W, ROWS_W)], zeros_v)
    pltpu.sync_copy(zeros_v, out_hbm.at[c, pl.ds(s * ROWS_W, ROWS_W)])


def _row_mask(i, cols):
    rows = i * BLK + lax.broadcasted_iota(jnp.int32, (BLK, cols), 0)
    return rows < N


def _vspec(shape):
    return pl.BlockSpec(shape, lambda i: tuple(0 for _ in shape))


def _rspec():
    return pl.BlockSpec((BLK, D), lambda i: (i, 0))


def _uspec():
    return pl.BlockSpec((2, BLK, HD), lambda i: (0, i, 0))


def _dinv(deg_ref):
    return lax.rsqrt(deg_ref[0] + deg_ref[1] + 1.0)


def _cat(r):
    return jnp.concatenate([r[0], r[1]], axis=1)


def _split_store(o_ref, v):
    o_ref[0] = v[:, :HD]
    o_ref[1] = v[:, HD:]


def _k_xw(x_ref, w_ref, b_ref, a_ref, o_ref):
    h = jnp.dot(x_ref[...], w_ref[...], preferred_element_type=jnp.float32)
    h = h + b_ref[...]
    a = a_ref[0, 0]
    o_ref[...] = jnp.where(h >= 0, h, a * h)


def _k_stats(h_ref, o_ref):
    i = pl.program_id(0)
    m = _row_mask(i, D)
    h = jnp.where(m, h_ref[...], 0.0)
    sums = jnp.concatenate(
        [jnp.sum(h, axis=0, keepdims=True),
         jnp.sum(h * h, axis=0, keepdims=True)], axis=0)

    @pl.when(i == 0)
    def _():
        o_ref[...] = jnp.zeros_like(o_ref)

    o_ref[...] += sums


def _k_apply(h_ref, st_ref, deg_ref, g_ref, b_ref, o_ref):
    i = pl.program_id(0)
    mean = st_ref[0:1] * (1.0 / N)
    var = st_ref[1:2] * (1.0 / N) - mean * mean
    h = (h_ref[...] - mean) * lax.rsqrt(var + EPS) * g_ref[...] + b_ref[...]
    u = jnp.where(_row_mask(i, D), _dinv(deg_ref) * h, 0.0)
    _split_store(o_ref, u)


def _k_mid(s_ref, u_ref, deg_ref, w1_ref, b1_ref, a2_ref, w2_ref, o_ref):
    i = pl.program_id(0)
    dinv = _dinv(deg_ref)
    agg = dinv * (_cat(s_ref) + _cat(u_ref))
    h = jnp.dot(agg, w1_ref[...], preferred_element_type=jnp.float32)
    h = h + b1_ref[...]
    a2 = a2_ref[0, 0]
    h = jnp.where(h >= 0, h, a2 * h)
    v = jnp.dot(h, w2_ref[...], preferred_element_type=jnp.float32)
    u = jnp.where(_row_mask(i, D), dinv * v, 0.0)
    _split_store(o_ref, u)


def _k_blkpre(s_ref, u_ref, deg_ref, b2_ref, a_ref, o_ref):
    h = _dinv(deg_ref) * (_cat(s_ref) + _cat(u_ref)) + b2_ref[...]
    a = a_ref[0, 0]
    o_ref[...] = jnp.where(h >= 0, h, a * h)


def _k_pool(s_ref, u_ref, deg_ref, b2_ref, batch_ref, p_ref, cnt_ref):
    i = pl.program_id(0)
    h = _dinv(deg_ref) * (_cat(s_ref) + _cat(u_ref)) + b2_ref[...]
    onehot = (batch_ref[...] ==
              lax.broadcasted_iota(jnp.int32, (BLK, G), 1)).astype(jnp.float32)
    p = lax.dot_general(onehot, h, (((0,), (0,)), ((), ())),
                        preferred_element_type=jnp.float32)
    ones = jnp.ones((BLK, D), jnp.float32)
    cnt = lax.dot_general(onehot, ones, (((0,), (0,)), ((), ())),
                          preferred_element_type=jnp.float32)

    @pl.when(i == 0)
    def _():
        p_ref[...] = jnp.zeros_like(p_ref)
        cnt_ref[...] = jnp.zeros_like(cnt_ref)

    p_ref[...] += p
    cnt_ref[...] += cnt


def _k_head(p_ref, cnt_ref, g_ref, b_ref, w1_ref, b1_ref, a_ref, w2_ref,
            b2_ref, o_ref):
    p = p_ref[...] / jnp.maximum(cnt_ref[...], 1.0)
    m = jnp.mean(p, axis=0, keepdims=True)
    v = jnp.mean((p - m) ** 2, axis=0, keepdims=True)
    p = (p - m) * lax.rsqrt(v + EPS) * g_ref[...] + b_ref[...]
    q = jnp.dot(p, w1_ref[...], preferred_element_type=jnp.float32)
    q = q + b1_ref[...]
    a = a_ref[0, 0]
    q = jnp.where(q >= 0, q, a * q)
    o_ref[...] = jnp.dot(q, w2_ref[...],
                         preferred_element_type=jnp.float32) + b2_ref[...]


def _call(body, grid, in_specs, out_specs, out_shape, *args):
    return pl.pallas_call(
        body, grid=(grid,), in_specs=in_specs, out_specs=out_specs,
        out_shape=out_shape)(*args)


def kernel(x, pre_w, pre_b, b0_a1, b0_bn_g, b0_bn_b, b0_w1, b0_b1, b0_a2,
           b0_w2, b0_b2, b1_a1, b1_bn_g, b1_bn_b, b1_w1, b1_b1, b1_a2, b1_w2,
           b1_b2, post_bn_g, post_bn_b, post_w1, post_b1, post_a, post_w2,
           post_b2, edge_index, batch):
    f32 = jnp.float32
    pad_e = jnp.full((EPAD - E,), N, jnp.int32)
    src2d = jnp.concatenate([edge_index[0].astype(jnp.int32), pad_e]).reshape(
        EROWS, 128)
    dst2d = jnp.concatenate([edge_index[1].astype(jnp.int32), pad_e]).reshape(
        EROWS, 128)
    x_pad = jnp.concatenate([x, jnp.zeros((NPAD - N, D), f32)])
    batch_pad = jnp.concatenate(
        [batch.astype(jnp.int32), jnp.full((NPAD - N,), G, jnp.int32)]
    ).reshape(NPAD, 1)

    def s11(v):
        return jnp.asarray(v, f32).reshape(1, 1)

    def row(v):
        return jnp.asarray(v, f32).reshape(1, -1)

    vec_sh = jax.ShapeDtypeStruct((NPAD, D), f32)
    u_sh = jax.ShapeDtypeStruct((2, NPAD, HD), f32)

    deg2 = _sc_deg(dst2d, jnp.ones((128,), f32), jnp.zeros((ROWS_W,), f32))
    deg3 = deg2.reshape(2, NPAD, 1)
    dspec = pl.BlockSpec((2, BLK, 1), lambda i: (0, i, 0))

    h1 = _call(_k_xw, GRID,
               [_rspec(), _vspec((D, H)), _vspec((1, H)), _vspec((1, 1))],
               _rspec(), vec_sh, x_pad, pre_w, row(pre_b), s11(b0_a1))

    def bn_to_u(h, g, b):
        st = _call(_k_stats, GRID, [_rspec()],
                   pl.BlockSpec((2, D), lambda i: (0, 0)),
                   jax.ShapeDtypeStruct((2, D), f32), h)
        return _call(_k_apply, GRID,
                     [_rspec(), _vspec((2, D)), dspec, _vspec((1, D)),
                      _vspec((1, D))],
                     _uspec(), u_sh, h, st, deg3, row(g), row(b))

    def mid(s_parts, u, w1, b1, a2, w2):
        return _call(_k_mid, GRID,
                     [_uspec(), _uspec(), dspec, _vspec((H, 2 * H)),
                      _vspec((1, 2 * H)), _vspec((1, 1)), _vspec((2 * H, H))],
                     _uspec(), u_sh, s_parts, u, deg3, w1, row(b1), s11(a2),
                     w2)

    u1 = bn_to_u(h1, b0_bn_g, b0_bn_b)
    s1 = _sc_agg(u1, src2d, dst2d)
    u2 = mid(s1, u1, b0_w1, b0_b1, b0_a2, b0_w2)
    s2 = _sc_agg(u2, src2d, dst2d)
    h5 = _call(_k_blkpre, GRID,
               [_uspec(), _uspec(), dspec, _vspec((1, D)), _vspec((1, 1))],
               _rspec(), vec_sh, s2, u2, deg3, row(b0_b2), s11(b1_a1))
    u3 = bn_to_u(h5, b1_bn_g, b1_bn_b)
    s3 = _sc_agg(u3, src2d, dst2d)
    u4 = mid(s3, u3, b1_w1, b1_b1, b1_a2, b1_w2)
    s4 = _sc_agg(u4, src2d, dst2d)

    p, cnt = _call(
        _k_pool, GRID,
        [_uspec(), _uspec(), dspec, _vspec((1, D)),
         pl.BlockSpec((BLK, 1), lambda i: (i, 0))],
        (pl.BlockSpec((G, D), lambda i: (0, 0)),
         pl.BlockSpec((G, D), lambda i: (0, 0))),
        (jax.ShapeDtypeStruct((G, D), f32), jax.ShapeDtypeStruct((G, D), f32)),
        s4, u4, deg3, row(b1_b2), batch_pad)

    return _call(_k_head, 1,
                 [_vspec((G, D)), _vspec((G, D)), _vspec((1, D)),
                  _vspec((1, D)), _vspec((H, 4 * H)), _vspec((1, 4 * H)),
                  _vspec((1, 1)), _vspec((4 * H, C)), _vspec((1, C))],
                 _vspec((G, C)), jax.ShapeDtypeStruct((G, C), f32),
                 p, cnt, row(post_bn_g), row(post_bn_b), post_w1,
                 row(post_b1), s11(post_a), post_w2, row(post_b2))


# speedup vs baseline: 1.0726x; 1.0726x over previous
import functools

import jax
import jax.numpy as jnp
from jax import lax
from jax.experimental import pallas as pl
from jax.experimental.pallas import tpu as pltpu
from jax.experimental.pallas import tpu_sc as plsc

N = 10000
E = 320000
D = 128
H = 128
G = 16
C = 10
EPS = 1e-5
HD = D // 2

NPAD = 10240
EPAD = 327680
EROWS = EPAD // 128
CHUNKS_W = EROWS // 16
DEG_CH_W = EROWS // 32
ROWS_W = 640
BLK = 1280
GRID = NPAD // BLK
NBUF = 2

_mesh = plsc.VectorSubcoreMesh(core_axis_name="c", subcore_axis_name="s")


@functools.partial(
    pl.kernel,
    out_type=jax.ShapeDtypeStruct((2, NPAD, HD), jnp.float32),
    mesh=_mesh,
    scratch_types=[
        pltpu.VMEM((CHUNKS_W, 128), jnp.int32),
        pltpu.VMEM((CHUNKS_W, 128), jnp.int32),
    ] + [pltpu.VMEM((128, HD), jnp.float32) for _ in range(2 * NBUF)] + [
        pltpu.VMEM_SHARED((NPAD, HD), jnp.float32),
        pltpu.SemaphoreType.DMA,
        pltpu.SemaphoreType.DMA,
    ],
    compiler_params=pltpu.CompilerParams(use_tc_tiling_on_sc=False),
)
def _sc_agg(u_hbm, src_hbm, dst_hbm, out_hbm, src_v, dst_v, b0, b1, b2, b3,
            acc, sem_g, sem_s):
    bufs = (b0, b1, b2, b3)
    c = lax.axis_index("c")
    s = lax.axis_index("s")
    pltpu.sync_copy(src_hbm.at[pl.ds(s * CHUNKS_W, CHUNKS_W)], src_v)
    pltpu.sync_copy(dst_hbm.at[pl.ds(s * CHUNKS_W, CHUNKS_W)], dst_v)
    pltpu.sync_copy(u_hbm.at[c, pl.ds(N + 48, 128)], bufs[0])
    rbase = s * ROWS_W
    for i in range(ROWS_W // 128):
        pltpu.sync_copy(bufs[0], acc.at[pl.ds(rbase + i * 128, 128)])
    for b in range(NBUF):
        pltpu.async_copy(u_hbm.at[c].at[src_v.at[b]], bufs[b], sem_g)
    plsc.subcore_barrier()

    def group(g, carry):
        for b in range(2 * NBUF):
            j = g * 2 * NBUF + b
            pltpu.make_async_copy(u_hbm.at[c].at[src_v.at[j]], bufs[b],
                                  sem_g).wait()
            pltpu.async_copy(bufs[b], acc.at[dst_v.at[j]], sem_s, add=True)
            prev = j - NBUF
            pb = (b + NBUF) % (2 * NBUF)

            @pl.when(prev >= 0)
            def _():
                pltpu.make_async_copy(bufs[pb], acc.at[dst_v.at[prev]],
                                      sem_s).wait()

            nxt = j + NBUF

            @pl.when(nxt < CHUNKS_W)
            def _():
                pltpu.async_copy(u_hbm.at[c].at[src_v.at[nxt]], bufs[pb],
                                 sem_g)
        return carry

    lax.fori_loop(0, CHUNKS_W // (2 * NBUF), group, 0)
    for k in range(NBUF):
        j = CHUNKS_W - NBUF + k
        pltpu.make_async_copy(bufs[(j % (2 * NBUF))],
                              acc.at[dst_v.at[j]], sem_s).wait()
    plsc.subcore_barrier()
    for i in range(ROWS_W // 128):
        pltpu.sync_copy(acc.at[pl.ds(rbase + i * 128, 128)], bufs[0])
        pltpu.sync_copy(bufs[0], out_hbm.at[c, pl.ds(rbase + i * 128, 128)])


@functools.partial(
    pl.kernel,
    out_type=jax.ShapeDtypeStruct((2, NPAD), jnp.float32),
    mesh=_mesh,
    scratch_types=[
        pltpu.VMEM((DEG_CH_W, 128), jnp.int32),
        pltpu.VMEM((128,), jnp.float32),
        pltpu.VMEM((ROWS_W,), jnp.float32),
        pltpu.VMEM_SHARED((NPAD,), jnp.float32),
    ],
)
def _sc_deg(dst_hbm, ones_hbm, zeros_hbm, out_hbm, dst_v, ones_v, zeros_v,
            acc):
    c = lax.axis_index("c")
    s = lax.axis_index("s")
    w = c * 16 + s
    pltpu.sync_copy(dst_hbm.at[pl.ds(w * DEG_CH_W, DEG_CH_W)], dst_v)
    pltpu.sync_copy(ones_hbm, ones_v)
    pltpu.sync_copy(zeros_hbm, zeros_v)
    pltpu.sync_copy(zeros_v, acc.at[pl.ds(s * ROWS_W, ROWS_W)])
    plsc.subcore_barrier()

    def body(j, carry):
        pltpu.sync_copy(ones_v, acc.at[dst_v.at[j]], add=True)
        return carry

    lax.fori_loop(0, DEG_CH_W, body, 0)
    plsc.subcore_barrier()
    pltpu.sync_copy(acc.at[pl.ds(s * ROWS_W, ROWS_W)], zeros_v)
    pltpu.sync_copy(zeros_v, out_hbm.at[c, pl.ds(s * ROWS_W, ROWS_W)])


def _row_mask(i, cols):
    rows = i * BLK + lax.broadcasted_iota(jnp.int32, (BLK, cols), 0)
    return rows < N


def _vspec(shape):
    return pl.BlockSpec(shape, lambda i: tuple(0 for _ in shape))


def _rspec():
    return pl.BlockSpec((BLK, D), lambda i: (i, 0))


def _uspec():
    return pl.BlockSpec((2, BLK, HD), lambda i: (0, i, 0))


def _dinv(deg_ref):
    return lax.rsqrt(deg_ref[0] + deg_ref[1] + 1.0)


def _cat(r):
    return jnp.concatenate([r[0], r[1]], axis=1)


def _split_store(o_ref, v):
    o_ref[0] = v[:, :HD]
    o_ref[1] = v[:, HD:]


def _acc_stats(i, h, o_ref):
    h = jnp.where(_row_mask(i, D), h, 0.0)
    sums = jnp.concatenate(
        [jnp.sum(h, axis=0, keepdims=True),
         jnp.sum(h * h, axis=0, keepdims=True)], axis=0)

    @pl.when(i == 0)
    def _():
        o_ref[...] = jnp.zeros_like(o_ref)

    o_ref[...] += sums


def _k_xw_stats(x_ref, w_ref, b_ref, a_ref, o_ref, st_ref):
    i = pl.program_id(0)
    h = jnp.dot(x_ref[...], w_ref[...], preferred_element_type=jnp.float32)
    h = h + b_ref[...]
    a = a_ref[0, 0]
    h = jnp.where(h >= 0, h, a * h)
    o_ref[...] = h
    _acc_stats(i, h, st_ref)


def _k_apply(h_ref, st_ref, deg_ref, g_ref, b_ref, o_ref):
    i = pl.program_id(0)
    mean = st_ref[0:1] * (1.0 / N)
    var = st_ref[1:2] * (1.0 / N) - mean * mean
    h = (h_ref[...] - mean) * lax.rsqrt(var + EPS) * g_ref[...] + b_ref[...]
    u = jnp.where(_row_mask(i, D), _dinv(deg_ref) * h, 0.0)
    _split_store(o_ref, u)


def _k_mid(s_ref, u_ref, deg_ref, w1_ref, b1_ref, a2_ref, w2_ref, o_ref):
    i = pl.program_id(0)
    dinv = _dinv(deg_ref)
    agg = dinv * (_cat(s_ref) + _cat(u_ref))
    h = jnp.dot(agg, w1_ref[...], preferred_element_type=jnp.float32)
    h = h + b1_ref[...]
    a2 = a2_ref[0, 0]
    h = jnp.where(h >= 0, h, a2 * h)
    v = jnp.dot(h, w2_ref[...], preferred_element_type=jnp.float32)
    u = jnp.where(_row_mask(i, D), dinv * v, 0.0)
    _split_store(o_ref, u)


def _k_blkpre_stats(s_ref, u_ref, deg_ref, b2_ref, a_ref, o_ref, st_ref):
    i = pl.program_id(0)
    h = _dinv(deg_ref) * (_cat(s_ref) + _cat(u_ref)) + b2_ref[...]
    a = a_ref[0, 0]
    h = jnp.where(h >= 0, h, a * h)
    o_ref[...] = h
    _acc_stats(i, h, st_ref)


def _k_pool(s_ref, u_ref, deg_ref, b2_ref, batch_ref, p_ref, cnt_ref):
    i = pl.program_id(0)
    h = _dinv(deg_ref) * (_cat(s_ref) + _cat(u_ref)) + b2_ref[...]
    onehot = (batch_ref[...] ==
              lax.broadcasted_iota(jnp.int32, (BLK, G), 1)).astype(jnp.float32)
    p = lax.dot_general(onehot, h, (((0,), (0,)), ((), ())),
                        preferred_element_type=jnp.float32)
    ones = jnp.ones((BLK, D), jnp.float32)
    cnt = lax.dot_general(onehot, ones, (((0,), (0,)), ((), ())),
                          preferred_element_type=jnp.float32)

    @pl.when(i == 0)
    def _():
        p_ref[...] = jnp.zeros_like(p_ref)
        cnt_ref[...] = jnp.zeros_like(cnt_ref)

    p_ref[...] += p
    cnt_ref[...] += cnt


def _k_head(p_ref, cnt_ref, g_ref, b_ref, w1_ref, b1_ref, a_ref, w2_ref,
            b2_ref, o_ref):
    p = p_ref[...] / jnp.maximum(cnt_ref[...], 1.0)
    m = jnp.mean(p, axis=0, keepdims=True)
    v = jnp.mean((p - m) ** 2, axis=0, keepdims=True)
    p = (p - m) * lax.rsqrt(v + EPS) * g_ref[...] + b_ref[...]
    q = jnp.dot(p, w1_ref[...], preferred_element_type=jnp.float32)
    q = q + b1_ref[...]
    a = a_ref[0, 0]
    q = jnp.where(q >= 0, q, a * q)
    o_ref[...] = jnp.dot(q, w2_ref[...],
                         preferred_element_type=jnp.float32) + b2_ref[...]


def _call(body, grid, in_specs, out_specs, out_shape, *args):
    return pl.pallas_call(
        body, grid=(grid,), in_specs=in_specs, out_specs=out_specs,
        out_shape=out_shape)(*args)


def kernel(x, pre_w, pre_b, b0_a1, b0_bn_g, b0_bn_b, b0_w1, b0_b1, b0_a2,
           b0_w2, b0_b2, b1_a1, b1_bn_g, b1_bn_b, b1_w1, b1_b1, b1_a2, b1_w2,
           b1_b2, post_bn_g, post_bn_b, post_w1, post_b1, post_a, post_w2,
           post_b2, edge_index, batch):
    f32 = jnp.float32
    pad_e = jnp.full((EPAD - E,), N, jnp.int32)
    src2d = jnp.concatenate([edge_index[0].astype(jnp.int32), pad_e]).reshape(
        EROWS, 128)
    dst2d = jnp.concatenate([edge_index[1].astype(jnp.int32), pad_e]).reshape(
        EROWS, 128)
    x_pad = jnp.concatenate([x, jnp.zeros((NPAD - N, D), f32)])
    batch_pad = jnp.concatenate(
        [batch.astype(jnp.int32), jnp.full((NPAD - N,), G, jnp.int32)]
    ).reshape(NPAD, 1)

    def s11(v):
        return jnp.asarray(v, f32).reshape(1, 1)

    def row(v):
        return jnp.asarray(v, f32).reshape(1, -1)

    vec_sh = jax.ShapeDtypeStruct((NPAD, D), f32)
    u_sh = jax.ShapeDtypeStruct((2, NPAD, HD), f32)

    deg2 = _sc_deg(dst2d, jnp.ones((128,), f32), jnp.zeros((ROWS_W,), f32))
    deg3 = deg2.reshape(2, NPAD, 1)
    dspec = pl.BlockSpec((2, BLK, 1), lambda i: (0, i, 0))

    st_spec = pl.BlockSpec((2, D), lambda i: (0, 0))
    st_sh = jax.ShapeDtypeStruct((2, D), f32)

    h1, st1 = _call(_k_xw_stats, GRID,
                    [_rspec(), _vspec((D, H)), _vspec((1, H)), _vspec((1, 1))],
                    (_rspec(), st_spec), (vec_sh, st_sh),
                    x_pad, pre_w, row(pre_b), s11(b0_a1))

    def bn_to_u(h, st, g, b):
        return _call(_k_apply, GRID,
                     [_rspec(), _vspec((2, D)), dspec, _vspec((1, D)),
                      _vspec((1, D))],
                     _uspec(), u_sh, h, st, deg3, row(g), row(b))

    def mid(s_parts, u, w1, b1, a2, w2):
        return _call(_k_mid, GRID,
                     [_uspec(), _uspec(), dspec, _vspec((H, 2 * H)),
                      _vspec((1, 2 * H)), _vspec((1, 1)), _vspec((2 * H, H))],
                     _uspec(), u_sh, s_parts, u, deg3, w1, row(b1), s11(a2),
                     w2)

    u1 = bn_to_u(h1, st1, b0_bn_g, b0_bn_b)
    s1 = _sc_agg(u1, src2d, dst2d)
    u2 = mid(s1, u1, b0_w1, b0_b1, b0_a2, b0_w2)
    s2 = _sc_agg(u2, src2d, dst2d)
    h5, st5 = _call(_k_blkpre_stats, GRID,
                    [_uspec(), _uspec(), dspec, _vspec((1, D)),
                     _vspec((1, 1))],
                    (_rspec(), st_spec), (vec_sh, st_sh),
                    s2, u2, deg3, row(b0_b2), s11(b1_a1))
    u3 = bn_to_u(h5, st5, b1_bn_g, b1_bn_b)
    s3 = _sc_agg(u3, src2d, dst2d)
    u4 = mid(s3, u3, b1_w1, b1_b1, b1_a2, b1_w2)
    s4 = _sc_agg(u4, src2d, dst2d)

    p, cnt = _call(
        _k_pool, GRID,
        [_uspec(), _uspec(), dspec, _vspec((1, D)),
         pl.BlockSpec((BLK, 1), lambda i: (i, 0))],
        (pl.BlockSpec((G, D), lambda i: (0, 0)),
         pl.BlockSpec((G, D), lambda i: (0, 0))),
        (jax.ShapeDtypeStruct((G, D), f32), jax.ShapeDtypeStruct((G, D), f32)),
        s4, u4, deg3, row(b1_b2), batch_pad)

    return _call(_k_head, 1,
                 [_vspec((G, D)), _vspec((G, D)), _vspec((1, D)),
                  _vspec((1, D)), _vspec((H, 4 * H)), _vspec((1, 4 * H)),
                  _vspec((1, 1)), _vspec((4 * H, C)), _vspec((1, C))],
                 _vspec((G, C)), jax.ShapeDtypeStruct((G, C), f32),
                 p, cnt, row(post_bn_g), row(post_bn_b), post_w1,
                 row(post_b1), s11(post_a), post_w2, row(post_b2))


# speedup vs baseline: 1.0771x; 1.0042x over previous
import functools

import jax
import jax.numpy as jnp
from jax import lax
from jax.experimental import pallas as pl
from jax.experimental.pallas import tpu as pltpu
from jax.experimental.pallas import tpu_sc as plsc

N = 10000
E = 320000
D = 128
H = 128
G = 16
C = 10
EPS = 1e-5
HD = D // 2

NPAD = 10240
EPAD = 327680
EROWS = EPAD // 128
CHUNKS_W = EROWS // 16
DEG_CH_W = EROWS // 32
ROWS_W = 640
BLK = 1280
GRID = NPAD // BLK
NBUF = 2

_mesh = plsc.VectorSubcoreMesh(core_axis_name="c", subcore_axis_name="s")


@functools.partial(
    pl.kernel,
    out_type=jax.ShapeDtypeStruct((2, NPAD, HD), jnp.float32),
    mesh=_mesh,
    scratch_types=[
        pltpu.VMEM((CHUNKS_W, 128), jnp.int32),
        pltpu.VMEM((CHUNKS_W, 128), jnp.int32),
    ] + [pltpu.VMEM((128, HD), jnp.float32) for _ in range(2 * NBUF)] + [
        pltpu.VMEM_SHARED((NPAD, HD), jnp.float32),
        pltpu.SemaphoreType.DMA,
        pltpu.SemaphoreType.DMA,
    ],
    compiler_params=pltpu.CompilerParams(use_tc_tiling_on_sc=False),
)
def _sc_agg(u_hbm, src_hbm, dst_hbm, out_hbm, src_v, dst_v, b0, b1, b2, b3,
            acc, sem_g, sem_s):
    bufs = (b0, b1, b2, b3)
    c = lax.axis_index("c")
    s = lax.axis_index("s")
    pltpu.sync_copy(src_hbm.at[pl.ds(s * CHUNKS_W, CHUNKS_W)], src_v)
    pltpu.sync_copy(dst_hbm.at[pl.ds(s * CHUNKS_W, CHUNKS_W)], dst_v)
    pltpu.sync_copy(u_hbm.at[c, pl.ds(N + 48, 128)], bufs[0])
    rbase = s * ROWS_W
    for i in range(ROWS_W // 128):
        pltpu.sync_copy(bufs[0], acc.at[pl.ds(rbase + i * 128, 128)])
    for b in range(NBUF):
        pltpu.async_copy(u_hbm.at[c].at[src_v.at[b]], bufs[b], sem_g)
    plsc.subcore_barrier()

    def group(g, carry):
        for b in range(2 * NBUF):
            j = g * 2 * NBUF + b
            pltpu.make_async_copy(u_hbm.at[c].at[src_v.at[j]], bufs[b],
                                  sem_g).wait()
            pltpu.async_copy(bufs[b], acc.at[dst_v.at[j]], sem_s, add=True)
            prev = j - NBUF
            pb = (b + NBUF) % (2 * NBUF)

            @pl.when(prev >= 0)
            def _():
                pltpu.make_async_copy(bufs[pb], acc.at[dst_v.at[prev]],
                                      sem_s).wait()

            nxt = j + NBUF

            @pl.when(nxt < CHUNKS_W)
            def _():
                pltpu.async_copy(u_hbm.at[c].at[src_v.at[nxt]], bufs[pb],
                                 sem_g)
        return carry

    lax.fori_loop(0, CHUNKS_W // (2 * NBUF), group, 0)
    for k in range(NBUF):
        j = CHUNKS_W - NBUF + k
        pltpu.make_async_copy(bufs[(j % (2 * NBUF))],
                              acc.at[dst_v.at[j]], sem_s).wait()
    plsc.subcore_barrier()
    nout = ROWS_W // 128
    for i in range(4):
        pltpu.async_copy(acc.at[pl.ds(rbase + i * 128, 128)],
                         bufs[i], sem_g)
    for i in range(nout):
        pltpu.make_async_copy(acc.at[pl.ds(rbase + i * 128, 128)],
                              bufs[i % 4], sem_g).wait()
        pltpu.async_copy(bufs[i % 4],
                         out_hbm.at[c, pl.ds(rbase + i * 128, 128)], sem_s)
        if i == 0:
            pltpu.make_async_copy(bufs[0],
                                  out_hbm.at[c, pl.ds(rbase, 128)],
                                  sem_s).wait()
            pltpu.async_copy(acc.at[pl.ds(rbase + 4 * 128, 128)],
                             bufs[0], sem_g)
    for i in range(1, nout):
        pltpu.make_async_copy(bufs[i % 4],
                              out_hbm.at[c, pl.ds(rbase + i * 128, 128)],
                              sem_s).wait()


@functools.partial(
    pl.kernel,
    out_type=jax.ShapeDtypeStruct((2, NPAD), jnp.float32),
    mesh=_mesh,
    scratch_types=[
        pltpu.VMEM((DEG_CH_W, 128), jnp.int32),
        pltpu.VMEM((128,), jnp.float32),
        pltpu.VMEM((ROWS_W,), jnp.float32),
        pltpu.VMEM_SHARED((NPAD,), jnp.float32),
    ],
)
def _sc_deg(dst_hbm, ones_hbm, zeros_hbm, out_hbm, dst_v, ones_v, zeros_v,
            acc):
    c = lax.axis_index("c")
    s = lax.axis_index("s")
    w = c * 16 + s
    pltpu.sync_copy(dst_hbm.at[pl.ds(w * DEG_CH_W, DEG_CH_W)], dst_v)
    pltpu.sync_copy(ones_hbm, ones_v)
    pltpu.sync_copy(zeros_hbm, zeros_v)
    pltpu.sync_copy(zeros_v, acc.at[pl.ds(s * ROWS_W, ROWS_W)])
    plsc.subcore_barrier()

    def body(j, carry):
        pltpu.sync_copy(ones_v, acc.at[dst_v.at[j]], add=True)
        return carry

    lax.fori_loop(0, DEG_CH_W, body, 0)
    plsc.subcore_barrier()
    pltpu.sync_copy(acc.at[pl.ds(s * ROWS_W, ROWS_W)], zeros_v)
    pltpu.sync_copy(zeros_v, out_hbm.at[c, pl.ds(s * ROWS_W, ROWS_W)])


def _row_mask(i, cols):
    rows = i * BLK + lax.broadcasted_iota(jnp.int32, (BLK, cols), 0)
    return rows < N


def _vspec(shape):
    return pl.BlockSpec(shape, lambda i: tuple(0 for _ in shape))


def _rspec():
    return pl.BlockSpec((BLK, D), lambda i: (i, 0))


def _uspec():
    return pl.BlockSpec((2, BLK, HD), lambda i: (0, i, 0))


def _dinv(deg_ref):
    return lax.rsqrt(deg_ref[0] + deg_ref[1] + 1.0)


def _cat(r):
    return jnp.concatenate([r[0], r[1]], axis=1)


def _split_store(o_ref, v):
    o_ref[0] = v[:, :HD]
    o_ref[1] = v[:, HD:]


def _acc_stats(i, h, o_ref):
    h = jnp.where(_row_mask(i, D), h, 0.0)
    sums = jnp.concatenate(
        [jnp.sum(h, axis=0, keepdims=True),
         jnp.sum(h * h, axis=0, keepdims=True)], axis=0)

    @pl.when(i == 0)
    def _():
        o_ref[...] = jnp.zeros_like(o_ref)

    o_ref[...] += sums


def _k_xw_stats(x_ref, w_ref, b_ref, a_ref, o_ref, st_ref):
    i = pl.program_id(0)
    h = jnp.dot(x_ref[...], w_ref[...], preferred_element_type=jnp.float32)
    h = h + b_ref[...]
    a = a_ref[0, 0]
    h = jnp.where(h >= 0, h, a * h)
    o_ref[...] = h
    _acc_stats(i, h, st_ref)


def _k_apply(h_ref, st_ref, deg_ref, g_ref, b_ref, o_ref):
    i = pl.program_id(0)
    mean = st_ref[0:1] * (1.0 / N)
    var = st_ref[1:2] * (1.0 / N) - mean * mean
    h = (h_ref[...] - mean) * lax.rsqrt(var + EPS) * g_ref[...] + b_ref[...]
    u = jnp.where(_row_mask(i, D), _dinv(deg_ref) * h, 0.0)
    _split_store(o_ref, u)


def _k_mid(s_ref, u_ref, deg_ref, w1_ref, b1_ref, a2_ref, w2_ref, o_ref):
    i = pl.program_id(0)
    dinv = _dinv(deg_ref)
    agg = dinv * (_cat(s_ref) + _cat(u_ref))
    h = jnp.dot(agg, w1_ref[...], preferred_element_type=jnp.float32)
    h = h + b1_ref[...]
    a2 = a2_ref[0, 0]
    h = jnp.where(h >= 0, h, a2 * h)
    v = jnp.dot(h, w2_ref[...], preferred_element_type=jnp.float32)
    u = jnp.where(_row_mask(i, D), dinv * v, 0.0)
    _split_store(o_ref, u)


def _k_blkpre_stats(s_ref, u_ref, deg_ref, b2_ref, a_ref, o_ref, st_ref):
    i = pl.program_id(0)
    h = _dinv(deg_ref) * (_cat(s_ref) + _cat(u_ref)) + b2_ref[...]
    a = a_ref[0, 0]
    h = jnp.where(h >= 0, h, a * h)
    o_ref[...] = h
    _acc_stats(i, h, st_ref)


def _k_pool(s_ref, u_ref, deg_ref, b2_ref, batch_ref, p_ref, cnt_ref):
    i = pl.program_id(0)
    h = _dinv(deg_ref) * (_cat(s_ref) + _cat(u_ref)) + b2_ref[...]
    onehot = (batch_ref[...] ==
              lax.broadcasted_iota(jnp.int32, (BLK, G), 1)).astype(jnp.float32)
    p = lax.dot_general(onehot, h, (((0,), (0,)), ((), ())),
                        preferred_element_type=jnp.float32)
    ones = jnp.ones((BLK, D), jnp.float32)
    cnt = lax.dot_general(onehot, ones, (((0,), (0,)), ((), ())),
                          preferred_element_type=jnp.float32)

    @pl.when(i == 0)
    def _():
        p_ref[...] = jnp.zeros_like(p_ref)
        cnt_ref[...] = jnp.zeros_like(cnt_ref)

    p_ref[...] += p
    cnt_ref[...] += cnt


def _k_head(p_ref, cnt_ref, g_ref, b_ref, w1_ref, b1_ref, a_ref, w2_ref,
            b2_ref, o_ref):
    p = p_ref[...] / jnp.maximum(cnt_ref[...], 1.0)
    m = jnp.mean(p, axis=0, keepdims=True)
    v = jnp.mean((p - m) ** 2, axis=0, keepdims=True)
    p = (p - m) * lax.rsqrt(v + EPS) * g_ref[...] + b_ref[...]
    q = jnp.dot(p, w1_ref[...], preferred_element_type=jnp.float32)
    q = q + b1_ref[...]
    a = a_ref[0, 0]
    q = jnp.where(q >= 0, q, a * q)
    o_ref[...] = jnp.dot(q, w2_ref[...],
                         preferred_element_type=jnp.float32) + b2_ref[...]


def _call(body, grid, in_specs, out_specs, out_shape, *args):
    return pl.pallas_call(
        body, grid=(grid,), in_specs=in_specs, out_specs=out_specs,
        out_shape=out_shape)(*args)


def kernel(x, pre_w, pre_b, b0_a1, b0_bn_g, b0_bn_b, b0_w1, b0_b1, b0_a2,
           b0_w2, b0_b2, b1_a1, b1_bn_g, b1_bn_b, b1_w1, b1_b1, b1_a2, b1_w2,
           b1_b2, post_bn_g, post_bn_b, post_w1, post_b1, post_a, post_w2,
           post_b2, edge_index, batch):
    f32 = jnp.float32
    pad_e = jnp.full((EPAD - E,), N, jnp.int32)
    src2d = jnp.concatenate([edge_index[0].astype(jnp.int32), pad_e]).reshape(
        EROWS, 128)
    dst2d = jnp.concatenate([edge_index[1].astype(jnp.int32), pad_e]).reshape(
        EROWS, 128)
    x_pad = jnp.concatenate([x, jnp.zeros((NPAD - N, D), f32)])
    batch_pad = jnp.concatenate(
        [batch.astype(jnp.int32), jnp.full((NPAD - N,), G, jnp.int32)]
    ).reshape(NPAD, 1)

    def s11(v):
        return jnp.asarray(v, f32).reshape(1, 1)

    def row(v):
        return jnp.asarray(v, f32).reshape(1, -1)

    vec_sh = jax.ShapeDtypeStruct((NPAD, D), f32)
    u_sh = jax.ShapeDtypeStruct((2, NPAD, HD), f32)

    deg2 = _sc_deg(dst2d, jnp.ones((128,), f32), jnp.zeros((ROWS_W,), f32))
    deg3 = deg2.reshape(2, NPAD, 1)
    dspec = pl.BlockSpec((2, BLK, 1), lambda i: (0, i, 0))

    st_spec = pl.BlockSpec((2, D), lambda i: (0, 0))
    st_sh = jax.ShapeDtypeStruct((2, D), f32)

    h1, st1 = _call(_k_xw_stats, GRID,
                    [_rspec(), _vspec((D, H)), _vspec((1, H)), _vspec((1, 1))],
                    (_rspec(), st_spec), (vec_sh, st_sh),
                    x_pad, pre_w, row(pre_b), s11(b0_a1))

    def bn_to_u(h, st, g, b):
        return _call(_k_apply, GRID,
                     [_rspec(), _vspec((2, D)), dspec, _vspec((1, D)),
                      _vspec((1, D))],
                     _uspec(), u_sh, h, st, deg3, row(g), row(b))

    def mid(s_parts, u, w1, b1, a2, w2):
        return _call(_k_mid, GRID,
                     [_uspec(), _uspec(), dspec, _vspec((H, 2 * H)),
                      _vspec((1, 2 * H)), _vspec((1, 1)), _vspec((2 * H, H))],
                     _uspec(), u_sh, s_parts, u, deg3, w1, row(b1), s11(a2),
                     w2)

    u1 = bn_to_u(h1, st1, b0_bn_g, b0_bn_b)
    s1 = _sc_agg(u1, src2d, dst2d)
    u2 = mid(s1, u1, b0_w1, b0_b1, b0_a2, b0_w2)
    s2 = _sc_agg(u2, src2d, dst2d)
    h5, st5 = _call(_k_blkpre_stats, GRID,
                    [_uspec(), _uspec(), dspec, _vspec((1, D)),
                     _vspec((1, 1))],
                    (_rspec(), st_spec), (vec_sh, st_sh),
                    s2, u2, deg3, row(b0_b2), s11(b1_a1))
    u3 = bn_to_u(h5, st5, b1_bn_g, b1_bn_b)
    s3 = _sc_agg(u3, src2d, dst2d)
    u4 = mid(s3, u3, b1_w1, b1_b1, b1_a2, b1_w2)
    s4 = _sc_agg(u4, src2d, dst2d)

    p, cnt = _call(
        _k_pool, GRID,
        [_uspec(), _uspec(), dspec, _vspec((1, D)),
         pl.BlockSpec((BLK, 1), lambda i: (i, 0))],
        (pl.BlockSpec((G, D), lambda i: (0, 0)),
         pl.BlockSpec((G, D), lambda i: (0, 0))),
        (jax.ShapeDtypeStruct((G, D), f32), jax.ShapeDtypeStruct((G, D), f32)),
        s4, u4, deg3, row(b1_b2), batch_pad)

    return _call(_k_head, 1,
                 [_vspec((G, D)), _vspec((G, D)), _vspec((1, D)),
                  _vspec((1, D)), _vspec((H, 4 * H)), _vspec((1, 4 * H)),
                  _vspec((1, 1)), _vspec((4 * H, C)), _vspec((1, C))],
                 _vspec((G, C)), jax.ShapeDtypeStruct((G, C), f32),
                 p, cnt, row(post_bn_g), row(post_bn_b), post_w1,
                 row(post_b1), s11(post_a), post_w2, row(post_b2))
